# Initial kernel scaffold; baseline (speedup 1.0000x reference)
import jax
import jax.numpy as jnp
from jax.experimental import pallas as pl


def kernel(x, adj_vals, edge_index, W1, b1, W2, b2):
    raise NotImplementedError("write your pallas kernel here")


# speedup vs baseline: 1.3542x; 1.3542x over previous
import functools

import jax
import jax.numpy as jnp
from jax import lax
from jax.experimental import pallas as pl
from jax.experimental.pallas import tpu as pltpu
from jax.experimental.pallas import tpu_sc as plsc

N = 10000
E = 160000
D = 256
HID = 512
NS = 16
NC = 2
L = 16
N_PAD = 10240
G = 84
BM = 512
NBLK = N_PAD // BM
BIAS_ROW = N


CW = 64
NQ = 4


def _make_spmm(n_out: int, n_table: int):
    rpt = n_out // NS
    chunk = rpt // 5
    zpt = N_PAD // NS

    def body(table, srcp, dstp, valp, out, src_v, dst_v, val_v, srcq_v,
             rows, acc, sem0, sem1):
        c = lax.axis_index("c")
        s = lax.axis_index("s")
        pltpu.sync_copy(srcp.at[s], src_v)
        pltpu.sync_copy(dstp.at[s], dst_v)
        pltpu.sync_copy(valp.at[s], val_v)

        four = jnp.full((L,), NQ, jnp.int32)

        def fix_idx(i, carry):
            for k in range(8):
                sl = pl.ds(k * L, L)
                src_v[i, sl] = src_v[i, sl] * four
            return carry

        lax.fori_loop(0, G, fix_idx, 0)

        sems = (sem0, sem1)
        zv = jnp.zeros((L,), jnp.float32)

        for p in range(2):
            q = 2 * p + c
            qvec = jnp.full((L,), q, jnp.int32)

            def mk_idx(i, carry):
                for k in range(8):
                    sl = pl.ds(k * L, L)
                    srcq_v[i, sl] = src_v[i, sl] + qvec
                return carry

            lax.fori_loop(0, G, mk_idx, 0)

            def zero_buf(i, carry):
                for k in range(CW // L):
                    rows[0, i, pl.ds(k * L, L)] = zv
                return carry

            lax.fori_loop(0, 128, zero_buf, 0)
            for zi in range(zpt // 128):
                pltpu.sync_copy(rows.at[0],
                                acc.at[pl.ds(s * zpt + zi * 128, 128)])
            plsc.subcore_barrier()

            pltpu.async_copy(table.at[srcq_v.at[0]], rows.at[0], sem0)
            pltpu.async_copy(table.at[srcq_v.at[1]], rows.at[1], sem1)

            def group(gi, carry):
                for b in range(2):
                    g = gi * 2 + b
                    pltpu.make_async_copy(
                        table.at[srcq_v.at[g]], rows.at[b], sems[b]).wait()

                    def scale(j, cc):
                        vals16 = val_v[g, pl.ds(j * L, L)]
                        for lane in range(L):
                            e = j * L + lane
                            vv = lax.broadcast(vals16[lane], (L,))
                            for k in range(CW // L):
                                sl = pl.ds(k * L, L)
                                rows[b, e, sl] = rows[b, e, sl] * vv
                        return cc

                    lax.fori_loop(0, 8, scale, 0)
                    pltpu.sync_copy(rows.at[b], acc.at[dst_v.at[g]],
                                    add=True)

                    @pl.when(g + 2 < G)
                    def _():
                        pltpu.async_copy(
                            table.at[srcq_v.at[g + 2]], rows.at[b], sems[b])
                return carry

            lax.fori_loop(0, G // 2, group, 0)
            plsc.subcore_barrier()
            for zi in range(5):
                r0 = s * rpt + zi * chunk
                pltpu.sync_copy(acc.at[pl.ds(r0, chunk)],
                                out.at[pl.ds(r0, chunk), q])
            plsc.subcore_barrier()

    return pl.kernel(
        body,
        out_type=jax.ShapeDtypeStruct((n_out, NQ, CW), jnp.float32),
        mesh=plsc.VectorSubcoreMesh(core_axis_name="c", subcore_axis_name="s"),
        compiler_params=pltpu.CompilerParams(use_tc_tiling_on_sc=False),
        scratch_types=[
            pltpu.VMEM((G, 128), jnp.int32),
            pltpu.VMEM((G, 128), jnp.int32),
            pltpu.VMEM((G, 128), jnp.float32),
            pltpu.VMEM((G, 128), jnp.int32),
            pltpu.VMEM((2, 128, CW), jnp.float32),
            pltpu.VMEM_SHARED((N_PAD, CW), jnp.float32),
            pltpu.SemaphoreType.DMA,
            pltpu.SemaphoreType.DMA,
        ],
    )


def _tc_body(ax_ref, w1_ref, b1_ref, w2_ref, b2_ref, out_ref):
    h = jnp.dot(ax_ref[...], w1_ref[...], preferred_element_type=jnp.float32)
    h = jnp.maximum(h + b1_ref[...], 0.0)
    out_ref[...] = jnp.dot(h, w2_ref[...],
                           preferred_element_type=jnp.float32)

    @pl.when(pl.program_id(0) == NBLK - 1)
    def _():
        out_ref[pl.ds(BIAS_ROW - (NBLK - 1) * BM, 1), :] = b2_ref[...]


_tc_call = pl.pallas_call(
    _tc_body,
    grid=(NBLK,),
    in_specs=[
        pl.BlockSpec((BM, D), lambda i: (i, 0)),
        pl.BlockSpec((D, HID), lambda i: (0, 0)),
        pl.BlockSpec((1, HID), lambda i: (0, 0)),
        pl.BlockSpec((HID, D), lambda i: (0, 0)),
        pl.BlockSpec((1, D), lambda i: (0, 0)),
    ],
    out_specs=pl.BlockSpec((BM, D), lambda i: (i, 0)),
    out_shape=jax.ShapeDtypeStruct((N_PAD, D), jnp.float32),
)


def _pack_edges(src, dst, vals):
    per = src.shape[0] // NS
    padn = G * 128 - per

    def p(a):
        return jnp.pad(a.reshape(NS, per), ((0, 0), (0, padn))).reshape(
            NS, G, 128)

    return p(src), p(dst), p(vals)


@functools.cache
def _spmm_calls():
    return _make_spmm(N_PAD, NQ * N), _make_spmm(N, NQ * N_PAD)


def kernel(x, adj_vals, edge_index, W1, b1, W2, b2):
    spmm1, spmm2 = _spmm_calls()
    src = edge_index[0].astype(jnp.int32)
    dst = edge_index[1].astype(jnp.int32)
    vals = adj_vals.astype(jnp.float32)

    s1, d1, v1 = _pack_edges(src, dst, vals)
    sb = jnp.concatenate([src, jnp.full((N,), BIAS_ROW, jnp.int32)])
    db = jnp.concatenate([dst, jnp.arange(N, dtype=jnp.int32)])
    vb = jnp.concatenate([vals, jnp.ones((N,), jnp.float32)])
    s2, d2, v2 = _pack_edges(sb, db, vb)

    ax = spmm1(x.reshape(NQ * N, CW), s1, d1, v1)
    g = _tc_call(ax.reshape(N_PAD, D), W1, b1.reshape(1, HID), W2,
                 b2.reshape(1, D))
    out = spmm2(g.reshape(NQ * N_PAD, CW), s2, d2, v2)
    return out.reshape(N, D)
